# baseline (speedup 1.0000x reference)
import functools

import jax
import jax.numpy as jnp
from jax import lax
from jax.experimental import pallas as pl
from jax.experimental.pallas import tpu as pltpu
from jax.experimental.pallas import tpu_sc as plsc

DIM = 64
NC = 2
NS = 16
NW = NC * NS
CHUNK = 512
SEG = 128
NBUF = 2


def _make_gather(B):
    assert B % (NW * CHUNK * NBUF) == 0
    b_per_w = B // NW
    n_chunks = b_per_w // CHUNK
    mesh = plsc.VectorSubcoreMesh(core_axis_name="c", subcore_axis_name="s")

    @functools.partial(
        pl.kernel,
        mesh=mesh,
        out_type=jax.ShapeDtypeStruct((B, DIM), jnp.float32),
        compiler_params=pltpu.CompilerParams(use_tc_tiling_on_sc=False),
        scratch_types=[
            pltpu.VMEM((NBUF, CHUNK), jnp.int32),
            pltpu.VMEM((NBUF, CHUNK, DIM), jnp.float32),
            pltpu.SemaphoreType.DMA,
            pltpu.SemaphoreType.DMA,
            pltpu.SemaphoreType.DMA,
            pltpu.SemaphoreType.DMA,
        ],
    )
    def gather_kernel(idx_hbm, table_hbm, out_hbm, idx_v, rows_v, g0, g1, o0, o1):
        gsem = (g0, g1)
        osem = (o0, o1)
        wid = lax.axis_index("s") * NC + lax.axis_index("c")
        wbase = wid * b_per_w

        def body(i, carry):
            handles = []
            for b in range(NBUF):
                base = wbase + (i * NBUF + b) * CHUNK

                @pl.when(i > 0)
                def _drain():
                    pltpu.make_async_copy(
                        rows_v.at[b], out_hbm.at[pl.ds(base, CHUNK)], osem[b]
                    ).wait()

                pltpu.sync_copy(idx_hbm.at[pl.ds(base, CHUNK)], idx_v.at[b])
                hs = []
                for j in range(CHUNK // SEG):
                    hs.append(
                        pltpu.async_copy(
                            table_hbm.at[idx_v.at[b, pl.ds(j * SEG, SEG)]],
                            rows_v.at[b, pl.ds(j * SEG, SEG)],
                            gsem[b],
                        )
                    )
                handles.append(hs)
            for b in range(NBUF):
                base = wbase + (i * NBUF + b) * CHUNK
                for h in handles[b]:
                    h.wait()
                pltpu.async_copy(
                    rows_v.at[b], out_hbm.at[pl.ds(base, CHUNK)], osem[b]
                )
            return carry

        lax.fori_loop(0, n_chunks // NBUF, body, 0)
        for b in range(NBUF):
            base = wbase + (n_chunks - NBUF + b) * CHUNK
            pltpu.make_async_copy(
                rows_v.at[b], out_hbm.at[pl.ds(base, CHUNK)], osem[b]
            ).wait()

    return gather_kernel


def kernel(input, weight):
    B, L = input.shape
    idx = input.reshape(-1).astype(jnp.int32)
    out = _make_gather(idx.shape[0])(idx, weight)
    return out.reshape(B, L, DIM)
